# baseline (speedup 1.0000x reference)
import functools

import jax
import jax.numpy as jnp
from jax import lax
from jax.experimental import pallas as pl
from jax.experimental.pallas import tpu as pltpu
from jax.experimental.pallas import tpu_sc as plsc

VOCAB = 1000
B, T = 1024, 200
N = B * T
NC, NS = 2, 16
NW = NC * NS
PER_W = N // NW
CHUNK = 40
NCHUNK = PER_W // CHUNK

_mesh = plsc.VectorSubcoreMesh(core_axis_name="c", subcore_axis_name="s")


@functools.partial(
    pl.kernel,
    out_type=jax.ShapeDtypeStruct((N, VOCAB), jnp.float32),
    mesh=_mesh,
    compiler_params=pltpu.CompilerParams(use_tc_tiling_on_sc=False),
    scratch_types=[
        pltpu.VMEM((PER_W,), jnp.int32),
        pltpu.VMEM((CHUNK, VOCAB), jnp.float32),
        pltpu.VMEM((CHUNK, VOCAB), jnp.float32),
        pltpu.SemaphoreType.DMA,
        pltpu.SemaphoreType.DMA,
        pltpu.SemaphoreType.DMA,
        pltpu.SemaphoreType.DMA,
    ],
)
def _gather_rows(idx_hbm, table_hbm, out_hbm,
                 idx_v, row0, row1, gsem0, gsem1, osem0, osem1):
    wid = lax.axis_index("s") * NC + lax.axis_index("c")
    base = wid * PER_W
    pltpu.sync_copy(idx_hbm.at[pl.ds(base, PER_W)], idx_v)

    bufs = (row0, row1)
    gsems = (gsem0, gsem1)
    osems = (osem0, osem1)

    def gather(i, b):
        return pltpu.make_async_copy(
            table_hbm.at[idx_v.at[pl.ds(i * CHUNK, CHUNK)]],
            bufs[b], gsems[b])

    def outcopy(i, b):
        return pltpu.make_async_copy(
            bufs[b], out_hbm.at[pl.ds(base + i * CHUNK, CHUNK)], osems[b])

    for b in range(2):
        gather(b, b).start()

    def body(p, carry):
        g = p * 2
        for b in range(2):
            i = g + b
            gather(i, b).wait()
            outcopy(i, b).start()

            @pl.when(i + 2 < NCHUNK)
            def _():
                outcopy(i, b).wait()
                gather(i + 2, b).start()
        return carry

    lax.fori_loop(0, NCHUNK // 2, body, 0)

    for b in range(2):
        outcopy(NCHUNK - 2 + b, b).wait()


def kernel(idx, token_embedding_table):
    flat = _gather_rows(idx.reshape(N).astype(jnp.int32),
                        token_embedding_table)
    return flat.reshape(B, T, VOCAB)
